# baseline (speedup 1.0000x reference)
import jax
import jax.numpy as jnp
from jax import lax
from jax.experimental import pallas as pl
from jax.experimental.pallas import tpu as pltpu
from jax.experimental.pallas import tpu_sc as plsc

B, L, D, V = 4096, 200, 512, 100000
NC, NS, LANES = 2, 16, 16
NW = NC * NS
BPW = B // NW
LC = 64
NCH = 3
NG = LC // LANES
LT = L - NCH * LC
LPAD = 256
DJ = D // LANES
EBLK = 8
S_CB0, S_CB1, S_T, S_BIAS, S_EMB, S_OUT = range(6)
GCMAX = BPW * NCH - 1


def _sc_body(embed_hbm, slf_hbm, w_hbm, bias_hbm, out_hbm,
             emb_v, idx_v, rows_v, rowst_v, out_v, sems):
    wid = lax.axis_index("s") * NC + lax.axis_index("c")
    b0 = wid * BPW
    lane = lax.iota(jnp.int32, LANES)
    masks = {k: (lane & k) == 0 for k in (1, 2, 4)}
    half = lane < 8
    dn = lax.GatherDimensionNumbers(offset_dims=(), collapsed_slice_dims=(0,),
                                    start_index_map=(0,))

    def dg(x, k):
        return lax.gather(x, (lane ^ k)[:, None], dn, (1,),
                          mode=lax.GatherScatterMode.PROMISE_IN_BOUNDS)

    pltpu.sync_copy(slf_hbm.at[pl.ds(b0 * L, BPW * L)], idx_v)

    def fire_chunk(g, w):
        rr = g // NCH
        cc = g - rr * NCH
        idx = idx_v.at[pl.ds(rr * L + cc * LC, LC)]
        pltpu.make_async_copy(w_hbm.at[idx], rows_v.at[w],
                              sems.at[S_CB0 + w]).start()

    def fire_tail(r):
        idxt = idx_v.at[pl.ds(r * L + NCH * LC, LT)]
        pltpu.make_async_copy(w_hbm.at[idxt], rowst_v.at[pl.ds(0, LT)],
                              sems.at[S_T]).start()

    def fire_bias_emb(r, p):
        for c in range(NCH):
            idx = idx_v.at[pl.ds(r * L + c * LC, LC)]
            pltpu.make_async_copy(bias_hbm.at[idx],
                                  out_v.at[p, pl.ds(c * LC, LC)],
                                  sems.at[S_BIAS]).start()
        idxt = idx_v.at[pl.ds(r * L + NCH * LC, LT)]
        pltpu.make_async_copy(bias_hbm.at[idxt],
                              out_v.at[p, pl.ds(NCH * LC, LT)],
                              sems.at[S_BIAS]).start()
        pltpu.make_async_copy(embed_hbm.at[b0 + r], emb_v.at[p],
                              sems.at[S_EMB]).start()

    def wait_bias(p):
        for c in range(NCH):
            pltpu.make_async_copy(bias_hbm.at[idx_v.at[pl.ds(0, LC)]],
                                  out_v.at[p, pl.ds(c * LC, LC)],
                                  sems.at[S_BIAS]).wait()
        pltpu.make_async_copy(bias_hbm.at[idx_v.at[pl.ds(0, LT)]],
                              out_v.at[p, pl.ds(NCH * LC, LT)],
                              sems.at[S_BIAS]).wait()

    def wait_emb(p):
        pltpu.make_async_copy(embed_hbm.at[b0], emb_v.at[p],
                              sems.at[S_EMB]).wait()

    def wait_chunk(w):
        pltpu.make_async_copy(w_hbm.at[idx_v.at[pl.ds(0, LC)]], rows_v.at[w],
                              sems.at[S_CB0 + w]).wait()

    def wait_tail():
        pltpu.make_async_copy(w_hbm.at[idx_v.at[pl.ds(0, LT)]],
                              rowst_v.at[pl.ds(0, LT)], sems.at[S_T]).wait()

    def wait_out(p, b):
        pltpu.make_async_copy(out_v.at[p], out_hbm.at[b], sems.at[S_OUT]).wait()

    def tree8(accs):
        cur = accs
        for k in (1, 2, 4):
            t = [c + dg(c, k) for c in cur]
            cur = [jnp.where(masks[k], t[2 * i], t[2 * i + 1])
                   for i in range(len(t) // 2)]
        z = cur[0]
        return z + dg(z, 8)

    def dot16(rows_ref, p, lbase, cdyn):
        halves = []
        for h in range(2):
            accs = [None] * 8
            for blk in range(DJ // EBLK):
                es = [emb_v[p, pl.ds((blk * EBLK + jj) * LANES, LANES)]
                      for jj in range(EBLK)]
                for i in range(8):
                    l = lbase + h * 8 + i
                    a = accs[i]
                    for jj in range(EBLK):
                        j = blk * EBLK + jj
                        if cdyn is None:
                            t = es[jj] * rows_ref[l, pl.ds(j * LANES, LANES)]
                        else:
                            t = es[jj] * rows_ref[cdyn, l, pl.ds(j * LANES, LANES)]
                        a = t if a is None else a + t
                    accs[i] = a
            halves.append(tree8(accs))
        return jnp.where(half, halves[0], halves[1])

    fire_chunk(jnp.int32(0), 0)
    fire_chunk(jnp.int32(1), 1)
    fire_tail(0)
    fire_bias_emb(0, 0)

    def row_body(r, _):
        b = b0 + r
        p = r & 1
        pn = 1 - p
        rn = jnp.minimum(r + 1, BPW - 1)

        @pl.when(r > 0)
        def _():
            wait_out(pn, b - 1)
        fire_bias_emb(rn, pn)
        wait_bias(p)
        wait_emb(p)

        def chunk_body(c, _):
            gc = r * NCH + c
            w = gc & 1
            wait_chunk(w)

            def g_body(g, _):
                off = c * LC + g * LANES
                out_v[p, pl.ds(off, LANES)] = (
                    out_v[p, pl.ds(off, LANES)] + dot16(rows_v, p, g * LANES, w))
                return 0
            lax.fori_loop(0, NG, g_body, 0)
            fire_chunk(jnp.minimum(gc + 2, GCMAX), w)
            return 0
        lax.fori_loop(0, NCH, chunk_body, 0)

        wait_tail()
        off = NCH * LC
        out_v[p, pl.ds(off, LANES)] = (
            out_v[p, pl.ds(off, LANES)] + dot16(rowst_v, p, 0, None))
        fire_tail(rn)
        pltpu.make_async_copy(out_v.at[p], out_hbm.at[b], sems.at[S_OUT]).start()
        return 0

    lax.fori_loop(0, BPW, row_body, 0)

    wait_bias(0)
    wait_emb(0)
    wait_chunk(0)
    wait_chunk(1)
    wait_tail()
    wait_out(1, b0 + BPW - 1)


@jax.jit
def _sparse_linear(embed, shortlist, weight, bias):
    mesh = plsc.VectorSubcoreMesh(
        core_axis_name="c", subcore_axis_name="s",
        num_cores=NC, num_subcores=NS)
    kfn = pl.kernel(
        _sc_body,
        out_type=jax.ShapeDtypeStruct((B, LPAD), jnp.float32),
        mesh=mesh,
        scratch_types=[
            pltpu.VMEM((2, D), jnp.float32),
            pltpu.VMEM((BPW * L,), jnp.int32),
            pltpu.VMEM((2, LC, D), jnp.float32),
            pltpu.VMEM((LANES, D), jnp.float32),
            pltpu.VMEM((2, LPAD), jnp.float32),
            pltpu.SemaphoreType.DMA((6,)),
        ],
    )
    return kfn(embed, shortlist, weight, bias)[:, :L]


def kernel(embed, shortlist, weight, bias):
    return _sparse_linear(embed, shortlist.astype(jnp.int32).reshape(B * L),
                          weight, bias.reshape(V))


# speedup vs baseline: 1.0403x; 1.0403x over previous
import jax
import jax.numpy as jnp
from jax import lax
from jax.experimental import pallas as pl
from jax.experimental.pallas import tpu as pltpu
from jax.experimental.pallas import tpu_sc as plsc

B, L, D, V = 4096, 200, 512, 100000
NC, NS, LANES = 2, 16, 16
NW = NC * NS
BPW = B // NW
LC = 64
NCH = 3
NG = LC // LANES
LT = L - NCH * LC
LPAD = 256
DJ = D // LANES
EBLK = 8
S_CB0, S_CB1, S_T, S_BIAS, S_EMB, S_OUT = range(6)
GCMAX = BPW * NCH - 1


def _sc_body(embed_hbm, slf_hbm, w_hbm, bias_hbm, out_hbm,
             emb_v, embu_v, idx_v, rows_a, rows_b, rowst_v, out_v, sems):
    wid = lax.axis_index("s") * NC + lax.axis_index("c")
    b0 = wid * BPW
    lane = lax.iota(jnp.int32, LANES)
    masks = {k: (lane & k) == 0 for k in (1, 2, 4)}
    half = lane < 8
    dn = lax.GatherDimensionNumbers(offset_dims=(), collapsed_slice_dims=(0,),
                                    start_index_map=(0,))

    def dg(x, k):
        return lax.gather(x, (lane ^ k)[:, None], dn, (1,),
                          mode=lax.GatherScatterMode.PROMISE_IN_BOUNDS)

    pltpu.sync_copy(slf_hbm.at[pl.ds(b0 * L, BPW * L)], idx_v)

    def fire_chunk_ref(g, ref, sem_i):
        rr = g // NCH
        cc = g - rr * NCH
        idx = idx_v.at[pl.ds(rr * L + cc * LC, LC)]
        pltpu.make_async_copy(w_hbm.at[idx], ref, sems.at[sem_i]).start()

    def fire_tail(r):
        idxt = idx_v.at[pl.ds(r * L + NCH * LC, LT)]
        pltpu.make_async_copy(w_hbm.at[idxt], rowst_v.at[pl.ds(0, LT)],
                              sems.at[S_T]).start()

    def fire_bias_emb(r, p):
        for c in range(NCH):
            idx = idx_v.at[pl.ds(r * L + c * LC, LC)]
            pltpu.make_async_copy(bias_hbm.at[idx],
                                  out_v.at[p, pl.ds(c * LC, LC)],
                                  sems.at[S_BIAS]).start()
        idxt = idx_v.at[pl.ds(r * L + NCH * LC, LT)]
        pltpu.make_async_copy(bias_hbm.at[idxt],
                              out_v.at[p, pl.ds(NCH * LC, LT)],
                              sems.at[S_BIAS]).start()
        pltpu.make_async_copy(embed_hbm.at[b0 + r], emb_v.at[p],
                              sems.at[S_EMB]).start()

    def wait_bias(p):
        for c in range(NCH):
            pltpu.make_async_copy(bias_hbm.at[idx_v.at[pl.ds(0, LC)]],
                                  out_v.at[p, pl.ds(c * LC, LC)],
                                  sems.at[S_BIAS]).wait()
        pltpu.make_async_copy(bias_hbm.at[idx_v.at[pl.ds(0, LT)]],
                              out_v.at[p, pl.ds(NCH * LC, LT)],
                              sems.at[S_BIAS]).wait()

    def wait_emb(p):
        pltpu.make_async_copy(embed_hbm.at[b0], emb_v.at[p],
                              sems.at[S_EMB]).wait()

    def wait_chunk_ref(ref, sem_i):
        pltpu.make_async_copy(w_hbm.at[idx_v.at[pl.ds(0, LC)]], ref,
                              sems.at[sem_i]).wait()

    def wait_tail():
        pltpu.make_async_copy(w_hbm.at[idx_v.at[pl.ds(0, LT)]],
                              rowst_v.at[pl.ds(0, LT)], sems.at[S_T]).wait()

    def wait_out(p, b):
        pltpu.make_async_copy(out_v.at[p], out_hbm.at[b], sems.at[S_OUT]).wait()

    def tree8(accs):
        cur = accs
        for k in (1, 2, 4):
            t = [c + dg(c, k) for c in cur]
            cur = [jnp.where(masks[k], t[2 * i], t[2 * i + 1])
                   for i in range(len(t) // 2)]
        z = cur[0]
        return z + dg(z, 8)

    def dot16(rows_ref, lbase):
        halves = []
        for h in range(2):
            accs = [None] * 8
            for blk in range(DJ // EBLK):
                es = [embu_v[pl.ds((blk * EBLK + jj) * LANES, LANES)]
                      for jj in range(EBLK)]
                for i in range(8):
                    l = lbase + h * 8 + i
                    ts = [es[jj] * rows_ref[l, pl.ds((blk * EBLK + jj) * LANES,
                                                     LANES)]
                          for jj in range(EBLK)]
                    q = ((ts[0] + ts[1]) + (ts[2] + ts[3])) +                         ((ts[4] + ts[5]) + (ts[6] + ts[7]))
                    accs[i] = q if accs[i] is None else accs[i] + q
            halves.append(tree8(accs))
        return jnp.where(half, halves[0], halves[1])

    fire_chunk_ref(jnp.int32(0), rows_a, S_CB0)
    fire_chunk_ref(jnp.int32(1), rows_b, S_CB1)
    fire_tail(0)
    fire_bias_emb(0, 0)

    def row_body(r, _):
        b = b0 + r
        p = r & 1
        pn = 1 - p
        rn = jnp.minimum(r + 1, BPW - 1)

        @pl.when(r > 0)
        def _():
            wait_out(pn, b - 1)
        fire_bias_emb(rn, pn)
        wait_bias(p)
        wait_emb(p)
        for j in range(DJ):
            embu_v[pl.ds(j * LANES, LANES)] = emb_v[p, pl.ds(j * LANES, LANES)]

        def chunk_body(c, _):
            gc = r * NCH + c
            w = gc & 1
            gnext = jnp.minimum(gc + 2, GCMAX)

            def run(ref, sem_i):
                wait_chunk_ref(ref, sem_i)

                def g_body(g, _):
                    off = c * LC + g * LANES
                    out_v[p, pl.ds(off, LANES)] = (
                        out_v[p, pl.ds(off, LANES)] + dot16(ref, g * LANES))
                    return 0
                lax.fori_loop(0, NG, g_body, 0)
                fire_chunk_ref(gnext, ref, sem_i)

            @pl.when(w == 0)
            def _():
                run(rows_a, S_CB0)

            @pl.when(w == 1)
            def _():
                run(rows_b, S_CB1)
            return 0
        lax.fori_loop(0, NCH, chunk_body, 0)

        wait_tail()
        off = NCH * LC
        out_v[p, pl.ds(off, LANES)] = (
            out_v[p, pl.ds(off, LANES)] + dot16(rowst_v, 0))
        fire_tail(rn)
        pltpu.make_async_copy(out_v.at[p], out_hbm.at[b], sems.at[S_OUT]).start()
        return 0

    lax.fori_loop(0, BPW, row_body, 0)

    wait_bias(0)
    wait_emb(0)
    wait_chunk_ref(rows_a, S_CB0)
    wait_chunk_ref(rows_b, S_CB1)
    wait_tail()
    wait_out(1, b0 + BPW - 1)


@jax.jit
def _sparse_linear(embed, shortlist, weight, bias):
    mesh = plsc.VectorSubcoreMesh(
        core_axis_name="c", subcore_axis_name="s",
        num_cores=NC, num_subcores=NS)
    kfn = pl.kernel(
        _sc_body,
        out_type=jax.ShapeDtypeStruct((B, LPAD), jnp.float32),
        mesh=mesh,
        scratch_types=[
            pltpu.VMEM((2, D), jnp.float32),
            pltpu.VMEM((D,), jnp.float32),
            pltpu.VMEM((BPW * L,), jnp.int32),
            pltpu.VMEM((LC, D), jnp.float32),
            pltpu.VMEM((LC, D), jnp.float32),
            pltpu.VMEM((LANES, D), jnp.float32),
            pltpu.VMEM((2, LPAD), jnp.float32),
            pltpu.SemaphoreType.DMA((6,)),
        ],
    )
    return kfn(embed, shortlist, weight, bias)[:, :L]


def kernel(embed, shortlist, weight, bias):
    return _sparse_linear(embed, shortlist.astype(jnp.int32).reshape(B * L),
                          weight, bias.reshape(V))
